# baseline (speedup 1.0000x reference)
import jax
import jax.numpy as jnp
from jax import lax
from jax.experimental import pallas as pl
from jax.experimental.pallas import tpu as pltpu
from jax.experimental.pallas import tpu_sc as plsc

D = 64
N = 16384
NC = 2
NS = 16
NW = NC * NS
PER_W = N // NW
PART = 128


def _sc_body(ids_hbm, w_hbm, table_hbm, out_hbm,
             idx_v, w_v, rows_v, part_v, gather_v, shared, sem):
    cid = lax.axis_index("c")
    sid = lax.axis_index("s")
    wid = sid * NC + cid

    pltpu.sync_copy(ids_hbm.at[pl.ds(wid * PER_W, PER_W)], idx_v)
    pltpu.sync_copy(w_hbm.at[pl.ds(wid * PER_W, PER_W)], w_v)

    def fire(c, carry):
        ids16 = idx_v[pl.ds(c * 16, 16)]
        base = c * 16
        for j in range(16):
            rid = ids16[j]
            pltpu.async_copy(table_hbm.at[pl.ds(rid * D, D)],
                             rows_v.at[pl.ds((base + j) * D, D)], sem)
        return carry

    lax.fori_loop(0, PER_W // 16, fire, 0)

    pltpu.make_async_copy(table_hbm.at[pl.ds(0, PER_W * D)], rows_v,
                          sem).wait()

    zero = jnp.zeros((16,), jnp.float32)

    def body(c, carry):
        a0, a1, a2, a3 = carry
        w_chunk = w_v[pl.ds(c * 16, 16)]
        base = c * 16
        for j in range(16):
            wsp = lax.gather(
                w_chunk, jnp.full((16, 1), j, jnp.int32),
                lax.GatherDimensionNumbers(offset_dims=(),
                                           collapsed_slice_dims=(0,),
                                           start_index_map=(0,)),
                slice_sizes=(1,),
                mode=lax.GatherScatterMode.PROMISE_IN_BOUNDS)
            r = (base + j) * D
            a0 = a0 + rows_v[pl.ds(r, 16)] * wsp
            a1 = a1 + rows_v[pl.ds(r + 16, 16)] * wsp
            a2 = a2 + rows_v[pl.ds(r + 32, 16)] * wsp
            a3 = a3 + rows_v[pl.ds(r + 48, 16)] * wsp
        return (a0, a1, a2, a3)

    a0, a1, a2, a3 = lax.fori_loop(0, PER_W // 16, body,
                                   (zero, zero, zero, zero))

    def wbody(c, acc):
        return acc + w_v[pl.ds(c * 16, 16)]

    wacc = lax.fori_loop(0, PER_W // 16, wbody, zero)

    part_v[pl.ds(0, 16)] = a0
    part_v[pl.ds(16, 16)] = a1
    part_v[pl.ds(32, 16)] = a2
    part_v[pl.ds(48, 16)] = a3
    part_v[pl.ds(64, 16)] = wacc
    part_v[pl.ds(80, 16)] = zero
    part_v[pl.ds(96, 16)] = zero
    part_v[pl.ds(112, 16)] = zero

    pltpu.sync_copy(part_v, shared.at[sid])
    plsc.subcore_barrier()

    @pl.when(sid == 0)
    def _():
        pltpu.sync_copy(shared, gather_v)
        for k in range(PART // 16):
            s = zero
            for r in range(NS):
                s = s + gather_v[r, pl.ds(k * 16, 16)]
            part_v[pl.ds(k * 16, 16)] = s
        pltpu.sync_copy(part_v, out_hbm.at[cid])


def kernel(ids, weights, table):
    ids_r = ids.astype(jnp.int32)
    table_r = table.reshape(-1)
    mesh = plsc.VectorSubcoreMesh(core_axis_name="c", subcore_axis_name="s")
    part = pl.kernel(
        _sc_body,
        mesh=mesh,
        out_type=jax.ShapeDtypeStruct((NC, PART), jnp.float32),
        scratch_types=[
            pltpu.VMEM((PER_W,), jnp.int32),
            pltpu.VMEM((PER_W,), jnp.float32),
            pltpu.VMEM((PER_W * D,), jnp.float32),
            pltpu.VMEM((PART,), jnp.float32),
            pltpu.VMEM((NS, PART), jnp.float32),
            pltpu.VMEM_SHARED((NS, PART), jnp.float32),
            pltpu.SemaphoreType.DMA,
        ],
    )(ids_r, weights, table_r)
    sums = part[:, :D].sum(axis=0)
    wsum = part[:, D:D + 16].sum()
    return sums / wsum


# speedup vs baseline: 3.6046x; 3.6046x over previous
import jax
import jax.numpy as jnp
from jax import lax
from jax.experimental import pallas as pl
from jax.experimental.pallas import tpu as pltpu
from jax.experimental.pallas import tpu_sc as plsc

D = 64
N = 16384
VOCAB_SZ = 1000000
NC = 2
NS = 16
NW = NC * NS
PER_W = N // NW
ZCH = 16384
TPW = 4 * ZCH
WPAD = NS * TPW
VB = 8192


def _sc_scatter(ids_hbm, w_hbm, wout_hbm, wsum_hbm,
                idx_v, w_v, zero_v, pv, shared_w):
    cid = lax.axis_index("c")
    sid = lax.axis_index("s")
    wid = sid * NC + cid

    pltpu.sync_copy(ids_hbm.at[pl.ds(wid * 4, 4)], idx_v)
    pltpu.sync_copy(w_hbm.at[pl.ds(wid * 4, 4)], w_v)

    zero = jnp.zeros((16,), jnp.float32)

    def zbody(i, carry):
        zero_v[pl.ds(i * 16, 16)] = zero
        return carry

    lax.fori_loop(0, ZCH // 16, zbody, 0)
    for t in range(TPW // ZCH):
        pltpu.sync_copy(zero_v, shared_w.at[pl.ds(sid * TPW + t * ZCH, ZCH)])
    plsc.subcore_barrier()

    for k in range(4):
        pltpu.sync_copy(w_v.at[k], shared_w.at[idx_v.at[k]], add=True)

    wacc = zero
    for k in range(4):
        for l in range(8):
            wacc = wacc + w_v[k, pl.ds(l * 16, 16)]
    pv[pl.ds(0, 16)] = wacc
    for k in range(1, 8):
        pv[pl.ds(k * 16, 16)] = zero
    pltpu.sync_copy(pv, wsum_hbm.at[wid])

    plsc.subcore_barrier()

    for t in range(TPW // ZCH):
        off = sid * TPW + t * ZCH
        pltpu.sync_copy(shared_w.at[pl.ds(off, ZCH)],
                        wout_hbm.at[cid, pl.ds(off, ZCH)])


def _tc_matvec(tt_ref, w_ref, out_ref):
    @pl.when(pl.program_id(0) == 0)
    def _():
        out_ref[...] = jnp.zeros_like(out_ref)

    v0 = pl.program_id(0) * VB
    vpos = v0 + lax.broadcasted_iota(jnp.int32, (D, VB), 1)
    tt = jnp.where(vpos < VOCAB_SZ, tt_ref[...], 0.0)
    ws = w_ref[0, :] + w_ref[1, :]
    out_ref[...] += jnp.sum(tt * ws[None, :], axis=1)[None, :]


def kernel(ids, weights, table):
    ids_r = ids.astype(jnp.int32).reshape(NW * 4, 128)
    w_r = weights.reshape(NW * 4, 128)
    table_t = table.T

    mesh = plsc.VectorSubcoreMesh(core_axis_name="c", subcore_axis_name="s")
    w_dense, w_parts = pl.kernel(
        _sc_scatter,
        mesh=mesh,
        out_type=[
            jax.ShapeDtypeStruct((NC, WPAD), jnp.float32),
            jax.ShapeDtypeStruct((NW, 128), jnp.float32),
        ],
        scratch_types=[
            pltpu.VMEM((4, 128), jnp.int32),
            pltpu.VMEM((4, 128), jnp.float32),
            pltpu.VMEM((ZCH,), jnp.float32),
            pltpu.VMEM((128,), jnp.float32),
            pltpu.VMEM_SHARED((WPAD,), jnp.float32),
        ],
    )(ids_r, w_r)

    out = pl.pallas_call(
        _tc_matvec,
        grid=((VOCAB_SZ + VB - 1) // VB,),
        in_specs=[
            pl.BlockSpec((D, VB), lambda i: (0, i)),
            pl.BlockSpec((NC, VB), lambda i: (0, i)),
        ],
        out_specs=pl.BlockSpec((1, D), lambda i: (0, 0)),
        out_shape=jax.ShapeDtypeStruct((1, D), jnp.float32),
        compiler_params=pltpu.CompilerParams(
            dimension_semantics=("arbitrary",)),
    )(table_t, w_dense)

    wsum = w_parts[:, :16].sum()
    return out[0] / wsum
